# baseline (speedup 1.0000x reference)
import functools

import jax
import jax.numpy as jnp
from jax import lax
from jax.experimental import pallas as pl
from jax.experimental.pallas import tpu as pltpu
from jax.experimental.pallas import tpu_sc as plsc

_NB = 4


@jax.jit
def _lookup(items, table):
    B, S = items.shape
    V, D = table.shape
    info = plsc.get_sparse_core_info()
    NC, NS = info.num_cores, info.num_subcores
    NW = NC * NS
    bw = B // NW
    n_groups = bw // _NB
    n_pairs = n_groups // 2
    slices = ((0, 128), (128, S - 128))
    mesh = plsc.VectorSubcoreMesh(core_axis_name="c", subcore_axis_name="s")

    @functools.partial(
        pl.kernel,
        out_type=jax.ShapeDtypeStruct((B, S, D), jnp.float32),
        mesh=mesh,
        scratch_types=[
            pltpu.VMEM((bw, S), jnp.int32),
            pltpu.VMEM((2, _NB, S, D), jnp.float32),
            pltpu.SemaphoreType.DMA,
            pltpu.SemaphoreType.DMA,
            pltpu.SemaphoreType.DMA,
            pltpu.SemaphoreType.DMA,
        ],
        compiler_params=pltpu.CompilerParams(use_tc_tiling_on_sc=False),
    )
    def gather_kernel(table_hbm, items_hbm, out_hbm, idx_v, rows_v,
                      gsem_a, gsem_b, ssem_a, ssem_b):
        wid = lax.axis_index("s") * NC + lax.axis_index("c")
        b0 = wid * bw
        pltpu.sync_copy(items_hbm.at[pl.ds(b0, bw)], idx_v)

        def gather_descs(g, p, sem):
            descs = []
            for j in range(_NB):
                for off, ln in slices:
                    descs.append(pltpu.make_async_copy(
                        table_hbm.at[idx_v.at[g * _NB + j, pl.ds(off, ln)]],
                        rows_v.at[p, j, pl.ds(off, ln)],
                        sem,
                    ))
            return descs

        def store_desc(g, p, sem):
            return pltpu.make_async_copy(
                rows_v.at[p],
                out_hbm.at[pl.ds(b0 + g * _NB, _NB)],
                sem,
            )

        for d in gather_descs(0, 0, gsem_a):
            d.start()

        def body(i, carry):
            g = 2 * i

            @pl.when(i > 0)
            def _():
                store_desc(g - 1, 1, ssem_b).wait()

            for d in gather_descs(g + 1, 1, gsem_b):
                d.start()
            for d in gather_descs(g, 0, gsem_a):
                d.wait()
            store_desc(g, 0, ssem_a).start()
            store_desc(g, 0, ssem_a).wait()

            @pl.when(i + 1 < n_pairs)
            def _():
                for d in gather_descs(g + 2, 0, gsem_a):
                    d.start()

            for d in gather_descs(g + 1, 1, gsem_b):
                d.wait()
            store_desc(g + 1, 1, ssem_b).start()
            return carry

        lax.fori_loop(0, n_pairs, body, 0)
        store_desc(n_groups - 1, 1, ssem_b).wait()

    return gather_kernel(table, items)


def kernel(items, table):
    return _lookup(items.astype(jnp.int32), table)


# speedup vs baseline: 1.2155x; 1.2155x over previous
import functools

import jax
import jax.numpy as jnp
from jax import lax
from jax.experimental import pallas as pl
from jax.experimental.pallas import tpu as pltpu
from jax.experimental.pallas import tpu_sc as plsc


@jax.jit
def _lookup(items_t, table):
    S, B = items_t.shape
    V, D = table.shape
    info = plsc.get_sparse_core_info()
    NC, NS, L = info.num_cores, info.num_subcores, info.num_lanes
    NW = NC * NS
    bw = B // NW
    n_pairs = S // 2
    DH = D // 8
    mesh = plsc.VectorSubcoreMesh(core_axis_name="c", subcore_axis_name="s")

    @functools.partial(
        pl.kernel,
        out_type=jax.ShapeDtypeStruct((S, DH, NW, 8, bw), jnp.float32),
        mesh=mesh,
        scratch_types=[
            pltpu.VMEM((S, bw), jnp.int32),
            pltpu.VMEM((2, bw, D), jnp.float32),
            pltpu.VMEM((2, D, bw + 1), jnp.float32),
            pltpu.SemaphoreType.DMA,
            pltpu.SemaphoreType.DMA,
            pltpu.SemaphoreType.DMA,
            pltpu.SemaphoreType.DMA,
        ],
        compiler_params=pltpu.CompilerParams(
            use_tc_tiling_on_sc=False, needs_layout_passes=False),
    )
    def gather_kernel(table_hbm, items_hbm, out_hbm, idx_v, gbuf, tbuf,
                      gsem_a, gsem_b, ssem_a, ssem_b):
        wid = lax.axis_index("s") * NC + lax.axis_index("c")
        pltpu.sync_copy(items_hbm.at[:, pl.ds(wid * bw, bw)], idx_v)

        gsems = (gsem_a, gsem_b)
        ssems = (ssem_a, ssem_b)

        def gather_desc(s, p):
            return pltpu.make_async_copy(
                table_hbm.at[idx_v.at[s]], gbuf.at[p], gsems[p])

        def store_descs(s, p):
            return [
                pltpu.make_async_copy(
                    tbuf.at[p, pl.ds(8 * g, 8), pl.ds(0, bw)],
                    out_hbm.at[s, g, wid],
                    ssems[p],
                )
                for g in range(DH)
            ]

        def transpose_patch(p):
            f_base = [lax.iota(jnp.int32, L) + 16 * k for k in range(D // L)]

            def row_body(l, carry):
                col = jnp.full((L,), l, dtype=jnp.int32)
                for k in range(D // L):
                    vals = gbuf[p, l, pl.ds(16 * k, L)]
                    plsc.store_scatter(tbuf.at[p], [f_base[k], col], vals)
                return carry

            lax.fori_loop(0, bw, row_body, 0)

        def phase(s, p):
            @pl.when(s + 1 < S)
            def _():
                gather_desc(s + 1, 1 - p).start()
            gather_desc(s, p).wait()
            @pl.when(s >= 2)
            def _():
                for d in store_descs(s - 2, p):
                    d.wait()
            transpose_patch(p)
            for d in store_descs(s, p):
                d.start()

        gather_desc(0, 0).start()

        def body(i, carry):
            phase(2 * i, 0)
            phase(2 * i + 1, 1)
            return carry

        lax.fori_loop(0, n_pairs, body, 0)
        for d in store_descs(S - 2, 0):
            d.wait()
        for d in store_descs(S - 1, 1):
            d.wait()

    return gather_kernel(table, items_t)


def kernel(items, table):
    out5 = _lookup(items.T.astype(jnp.int32), table)
    B, S = items.shape
    D = table.shape[1]
    t = jnp.transpose(out5, (2, 4, 0, 1, 3))
    return t.reshape(B, S, D)


# speedup vs baseline: 1.6362x; 1.3461x over previous
import functools

import jax
import jax.numpy as jnp
from jax import lax
from jax.experimental import pallas as pl
from jax.experimental.pallas import tpu as pltpu
from jax.experimental.pallas import tpu_sc as plsc


@jax.jit
def _lookup(items_t, table):
    S, B = items_t.shape
    V, D = table.shape
    info = plsc.get_sparse_core_info()
    NC, NS, L = info.num_cores, info.num_subcores, info.num_lanes
    NW = NC * NS
    bw = B // NW
    n_pairs = S // 2
    DH = D // 8
    mesh = plsc.VectorSubcoreMesh(core_axis_name="c", subcore_axis_name="s")

    @functools.partial(
        pl.kernel,
        out_type=jax.ShapeDtypeStruct((S, DH, NW, 8, bw), jnp.float32),
        mesh=mesh,
        scratch_types=[
            pltpu.VMEM((S, bw), jnp.int32),
            pltpu.VMEM((4, bw, D), jnp.float32),
            pltpu.VMEM((2, D, bw + 1), jnp.float32),
            pltpu.SemaphoreType.DMA,
            pltpu.SemaphoreType.DMA,
            pltpu.SemaphoreType.DMA,
            pltpu.SemaphoreType.DMA,
            pltpu.SemaphoreType.DMA,
            pltpu.SemaphoreType.DMA,
        ],
        compiler_params=pltpu.CompilerParams(
            use_tc_tiling_on_sc=False, needs_layout_passes=False),
    )
    def gather_kernel(table_hbm, items_hbm, out_hbm, idx_v, gbuf, tbuf,
                      gsem_a, gsem_b, gsem_c, gsem_d, ssem_a, ssem_b):
        wid = lax.axis_index("s") * NC + lax.axis_index("c")
        pltpu.sync_copy(items_hbm.at[:, pl.ds(wid * bw, bw)], idx_v)

        gsems = (gsem_a, gsem_b, gsem_c, gsem_d)
        ssems = (ssem_a, ssem_b)

        def gather_desc(s, p):
            return pltpu.make_async_copy(
                table_hbm.at[idx_v.at[s]], gbuf.at[p], gsems[p])

        def store_descs(s, p):
            return [
                pltpu.make_async_copy(
                    tbuf.at[p, pl.ds(8 * g, 8), pl.ds(0, bw)],
                    out_hbm.at[s, g, wid],
                    ssems[p],
                )
                for g in range(DH)
            ]

        def transpose_patch(gp, tp):
            f_base = [lax.iota(jnp.int32, L) + 16 * k for k in range(D // L)]

            @plsc.parallel_loop(0, bw, unroll=8)
            def row_body(l):
                col = jnp.full((L,), l, dtype=jnp.int32)
                for k in range(D // L):
                    vals = gbuf[gp, l, pl.ds(16 * k, L)]
                    plsc.store_scatter(tbuf.at[tp], [f_base[k], col], vals)

        def phase(s, j):
            @pl.when(s + 2 < S)
            def _():
                gather_desc(s + 2, (j + 2) % 4).start()
            @pl.when(s >= 2)
            def _():
                for d in store_descs(s - 2, j % 2):
                    d.wait()
            gather_desc(s, j).wait()
            transpose_patch(j, j % 2)
            for d in store_descs(s, j % 2):
                d.start()

        gather_desc(0, 0).start()
        gather_desc(1, 1).start()

        def body(i, carry):
            for j in range(4):
                phase(4 * i + j, j)
            return carry

        lax.fori_loop(0, S // 4, body, 0)
        for d in store_descs(S - 2, 0):
            d.wait()
        for d in store_descs(S - 1, 1):
            d.wait()

    return gather_kernel(table, items_t)


def kernel(items, table):
    out5 = _lookup(items.T.astype(jnp.int32), table)
    B, S = items.shape
    D = table.shape[1]
    t = jnp.transpose(out5, (2, 4, 0, 1, 3))
    return t.reshape(B, S, D)
